# baseline (device time: 12957 ns/iter reference)
import jax
import jax.numpy as jnp
from jax import lax
from jax.experimental import pallas as pl
from jax.experimental.pallas import tpu as pltpu

N_DEV = 4


def kernel(x, Wp):
    b, h_loc, w, c = x.shape
    c_out = Wp.shape[1]
    n_global = N_DEV * h_loc * w
    eps = 1e-5

    def body(x_ref, wp_ref, out_ref, comm_ref, xb_ref, send_sems, recv_sems):
        my_pos = lax.axis_index("i")

        barrier_sem = pltpu.get_barrier_semaphore()
        for d in range(1, N_DEV):
            peer = lax.rem(my_pos + d, N_DEV)
            pl.semaphore_signal(
                barrier_sem, inc=1,
                device_id=(peer,), device_id_type=pl.DeviceIdType.MESH,
            )
        pl.semaphore_wait(barrier_sem, N_DEV - 1)

        xv = x_ref[...]
        s1 = jnp.sum(xv, axis=(1, 2))
        s2 = jnp.sum(xv * xv, axis=(1, 2))
        comm_ref[0] = jnp.concatenate([s1, s2], axis=0)

        sends = []
        for d in range(1, N_DEV):
            peer = lax.rem(my_pos + d, N_DEV)
            slot = N_DEV - d
            rdma = pltpu.make_async_remote_copy(
                src_ref=comm_ref.at[0],
                dst_ref=comm_ref.at[slot],
                send_sem=send_sems.at[d - 1],
                recv_sem=recv_sems.at[slot - 1],
                device_id=(peer,),
                device_id_type=pl.DeviceIdType.MESH,
            )
            rdma.start()
            sends.append(rdma)

        xb_ref[...] = xv.astype(jnp.bfloat16)

        for o in range(1, N_DEV):
            recv = pltpu.make_async_remote_copy(
                src_ref=comm_ref.at[0],
                dst_ref=comm_ref.at[o],
                send_sem=send_sems.at[o - 1],
                recv_sem=recv_sems.at[o - 1],
                device_id=(my_pos,),
                device_id_type=pl.DeviceIdType.MESH,
            )
            recv.wait_recv()
        for rdma in sends:
            rdma.wait_send()

        total = comm_ref[0] + comm_ref[1] + comm_ref[2] + comm_ref[3]
        mean = total[:b] / n_global
        var = total[b:] / n_global - mean * mean
        inv = lax.rsqrt(var + eps)
        mean16 = mean.astype(jnp.bfloat16)
        inv16 = inv.astype(jnp.bfloat16)
        h = (xb_ref[...] - mean16[:, None, None, :]) * inv16[:, None, None, :]
        a = h * jax.nn.sigmoid(h)
        out = jnp.dot(
            a.reshape(b * h_loc * w, c),
            wp_ref[...].astype(jnp.bfloat16),
            preferred_element_type=jnp.float32,
        )
        out_ref[...] = out.reshape(b, h_loc, w, c_out)

    return pl.pallas_call(
        body,
        out_shape=jax.ShapeDtypeStruct((b, h_loc, w, c_out), jnp.float32),
        in_specs=[
            pl.BlockSpec(memory_space=pltpu.VMEM),
            pl.BlockSpec(memory_space=pltpu.VMEM),
        ],
        out_specs=pl.BlockSpec(memory_space=pltpu.VMEM),
        scratch_shapes=[
            pltpu.VMEM((N_DEV, 2 * b, c), jnp.float32),
            pltpu.VMEM((b, h_loc, w, c), jnp.bfloat16),
            pltpu.SemaphoreType.DMA((N_DEV - 1,)),
            pltpu.SemaphoreType.DMA((N_DEV - 1,)),
        ],
        compiler_params=pltpu.CompilerParams(collective_id=0),
    )(x, Wp)


# device time: 7274 ns/iter; 1.7813x vs baseline; 1.7813x over previous
import jax
import jax.numpy as jnp
from jax import lax
from jax.experimental import pallas as pl
from jax.experimental.pallas import tpu as pltpu

N_DEV = 4


def kernel(x, Wp):
    b, h_loc, w, c = x.shape
    c_out = Wp.shape[1]
    n_global = N_DEV * h_loc * w
    eps = 1e-5

    def body(x_ref, wp_ref, out_ref, comm_ref, send_sems, recv_sems):
        my_pos = lax.axis_index("i")

        barrier_sem = pltpu.get_barrier_semaphore()
        for d in range(1, N_DEV):
            peer = lax.rem(my_pos + d, N_DEV)
            pl.semaphore_signal(
                barrier_sem, inc=1,
                device_id=(peer,), device_id_type=pl.DeviceIdType.MESH,
            )
        pl.semaphore_wait(barrier_sem, N_DEV - 1)

        xv = x_ref[...]
        s1 = jnp.sum(xv, axis=(1, 2))
        s2 = jnp.sum(xv * xv, axis=(1, 2))
        comm_ref[0] = jnp.concatenate([s1, s2], axis=0)

        sends = []
        for d in range(1, N_DEV):
            peer = lax.rem(my_pos + d, N_DEV)
            slot = N_DEV - d
            rdma = pltpu.make_async_remote_copy(
                src_ref=comm_ref.at[0],
                dst_ref=comm_ref.at[slot],
                send_sem=send_sems.at[d - 1],
                recv_sem=recv_sems.at[slot - 1],
                device_id=(peer,),
                device_id_type=pl.DeviceIdType.MESH,
            )
            rdma.start()
            sends.append(rdma)

        for o in range(1, N_DEV):
            recv = pltpu.make_async_remote_copy(
                src_ref=comm_ref.at[0],
                dst_ref=comm_ref.at[o],
                send_sem=send_sems.at[o - 1],
                recv_sem=recv_sems.at[o - 1],
                device_id=(my_pos,),
                device_id_type=pl.DeviceIdType.MESH,
            )
            recv.wait_recv()
        for rdma in sends:
            rdma.wait_send()

        total = comm_ref[0] + comm_ref[1] + comm_ref[2] + comm_ref[3]
        mean = total[:b] / n_global
        var = total[b:] / n_global - mean * mean
        inv = lax.rsqrt(var + eps)
        h = (xv - mean[:, None, None, :]) * inv[:, None, None, :]
        a = h * jax.nn.sigmoid(h)
        out = jnp.dot(
            a.reshape(b * h_loc * w, c), wp_ref[...],
            preferred_element_type=jnp.float32,
        )
        out_ref[...] = out.reshape(b, h_loc, w, c_out)

    return pl.pallas_call(
        body,
        out_shape=jax.ShapeDtypeStruct((b, h_loc, w, c_out), jnp.float32),
        in_specs=[
            pl.BlockSpec(memory_space=pltpu.VMEM),
            pl.BlockSpec(memory_space=pltpu.VMEM),
        ],
        out_specs=pl.BlockSpec(memory_space=pltpu.VMEM),
        scratch_shapes=[
            pltpu.VMEM((N_DEV, 2 * b, c), jnp.float32),
            pltpu.SemaphoreType.DMA((N_DEV - 1,)),
            pltpu.SemaphoreType.DMA((N_DEV - 1,)),
        ],
        compiler_params=pltpu.CompilerParams(collective_id=0),
    )(x, Wp)
